# Initial kernel scaffold; baseline (speedup 1.0000x reference)
import jax
import jax.numpy as jnp
from jax.experimental import pallas as pl


def kernel(x, edge_index, W0, b0, gamma0, beta0, W1, b1):
    raise NotImplementedError("write your pallas kernel here")


# speedup vs baseline: 27.9040x; 27.9040x over previous
import dataclasses
import functools

import jax
import jax.numpy as jnp
from jax import lax
from jax.experimental import pallas as pl
from jax.experimental.pallas import tpu as pltpu
from jax.experimental.pallas import tpu_sc as plsc

N = 10000
NPAD = 10240
E = 320000
D_IN = 128
D_HID = 128
D_OUT = 64
LCS = 0.1

NW = 32
EPW = E // NW
CHUNK = 80
NCHUNK = EPW // CHUNK
RPS = NPAD // 16

_MESH = dict(core_axis_name="c", subcore_axis_name="s")

_SC_CP = pltpu.CompilerParams()
if "needs_layout_passes" in pltpu.CompilerParams.__dataclass_fields__:
    _SC_CP = dataclasses.replace(_SC_CP, needs_layout_passes=False)


@functools.partial(
    pl.kernel,
    out_type=jax.ShapeDtypeStruct((NW, NPAD), jnp.float32),
    mesh=plsc.VectorSubcoreMesh(**_MESH),
    compiler_params=_SC_CP,
    scratch_types=[
        pltpu.VMEM((EPW,), jnp.int32),
        pltpu.VMEM((NPAD,), jnp.float32),
    ],
)
def _sc_hist(dst_hbm, out_hbm, dst_v, hist_v):
    wid = lax.axis_index("s") * 2 + lax.axis_index("c")
    pltpu.sync_copy(dst_hbm.at[wid], dst_v)

    zeros16 = jnp.zeros((16,), jnp.float32)

    @pl.loop(0, NPAD // 16)
    def _(i):
        hist_v[pl.ds(i * 16, 16)] = zeros16

    ones16 = jnp.ones((16,), jnp.float32)

    @pl.loop(0, EPW // 16)
    def _(j):
        idx = dst_v[pl.ds(j * 16, 16)]
        plsc.addupdate_scatter(hist_v, [idx], ones16)

    pltpu.sync_copy(hist_v, out_hbm.at[wid])


def _make_segsum(d):
    @functools.partial(
        pl.kernel,
        out_type=jax.ShapeDtypeStruct((2, NPAD, d), jnp.float32),
        mesh=plsc.VectorSubcoreMesh(**_MESH),
        scratch_types=[
            pltpu.VMEM((NCHUNK, CHUNK), jnp.int32),
            pltpu.VMEM((NCHUNK, CHUNK), jnp.int32),
            pltpu.VMEM((CHUNK, d), jnp.float32),
            pltpu.VMEM_SHARED((NPAD, d), jnp.float32),
        ],
    )
    def _sc_segsum(y_hbm, src_hbm, dst_hbm, zeros_hbm, out_hbm,
                   src_v, dst_v, rows_v, acc_s):
        core = lax.axis_index("c")
        sid = lax.axis_index("s")
        wid = sid * 2 + core
        base = sid * RPS

        pltpu.sync_copy(src_hbm.at[wid], src_v)
        pltpu.sync_copy(dst_hbm.at[wid], dst_v)

        pltpu.sync_copy(zeros_hbm, rows_v)

        @pl.loop(0, RPS // CHUNK)
        def _(k):
            pltpu.sync_copy(rows_v, acc_s.at[pl.ds(base + k * CHUNK, CHUNK)])

        plsc.subcore_barrier()

        @pl.loop(0, NCHUNK)
        def _(j):
            pltpu.sync_copy(y_hbm.at[src_v.at[j]], rows_v)
            pltpu.sync_copy(rows_v, acc_s.at[dst_v.at[j]], add=True)

        plsc.subcore_barrier()

        @pl.loop(0, RPS // CHUNK)
        def _(k):
            sl = pl.ds(base + k * CHUNK, CHUNK)
            pltpu.sync_copy(acc_s.at[sl], rows_v)
            pltpu.sync_copy(rows_v, out_hbm.at[core, sl])

    return _sc_segsum


_sc_segsum_hid = _make_segsum(D_HID)


BLK = 2048
NBLK = NPAD // BLK

_row_spec = pl.BlockSpec((BLK, D_HID), lambda i: (i, 0))
_col_spec = pl.BlockSpec((BLK, 1), lambda i: (i, 0))
_vec128_spec = pl.BlockSpec((D_HID,), lambda i: (0,))


def _tc_a_body(hist_ref, x_ref, w0_ref, xw_ref, y0_ref, indeg_ref):
    ones = jnp.ones((NW, 1), jnp.float32)
    indeg = lax.dot_general(hist_ref[...], ones, (((0,), (0,)), ((), ())),
                            preferred_element_type=jnp.float32)
    dinv = lax.rsqrt(indeg + 1.0)
    xw = jnp.dot(x_ref[...], w0_ref[...],
                 preferred_element_type=jnp.float32,
                 precision=lax.Precision.HIGHEST)
    xw_ref[...] = xw
    y0_ref[...] = xw * dinv
    indeg_ref[...] = indeg


def _tc_a(hist, x_pad, w0):
    return pl.pallas_call(
        _tc_a_body,
        grid=(NBLK,),
        in_specs=[
            pl.BlockSpec((NW, BLK), lambda i: (0, i)),
            _row_spec,
            pl.BlockSpec((D_IN, D_HID), lambda i: (0, 0)),
        ],
        out_specs=[_row_spec, _row_spec, _col_spec],
        out_shape=[
            jax.ShapeDtypeStruct((NPAD, D_HID), jnp.float32),
            jax.ShapeDtypeStruct((NPAD, D_HID), jnp.float32),
            jax.ShapeDtypeStruct((NPAD, 1), jnp.float32),
        ],
    )(hist, x_pad, w0)


def _tc_b1_body(p0_ref, p1_ref, xw_ref, indeg_ref, b0_ref, g0_ref, be0_ref,
                h_ref, smin_ref, smax_ref):
    i = pl.program_id(0)
    indeg = indeg_ref[...]
    dinv = lax.rsqrt(indeg + 1.0)
    agg = ((p0_ref[...] + p1_ref[...]) * dinv
           + dinv * dinv * xw_ref[...] + b0_ref[...][None, :])
    mu = jnp.mean(agg, axis=1, keepdims=True)
    cen = agg - mu
    var = jnp.mean(cen * cen, axis=1, keepdims=True)
    h = cen * lax.rsqrt(var + 1e-5) * g0_ref[...][None, :] + be0_ref[...][None, :]
    h = jnp.maximum(h, 0.0)
    h_ref[...] = h

    scores = jnp.sqrt(jnp.sum(h * h, axis=1, keepdims=True))
    rows = i * BLK + lax.broadcasted_iota(jnp.int32, (BLK, 1), 0)
    valid = rows < N
    big = jnp.float32(3e38)
    smin_ref[...] = jnp.min(jnp.where(valid, scores, big), keepdims=True)[None]
    smax_ref[...] = jnp.max(jnp.where(valid, scores, -big), keepdims=True)[None]


def _tc_b1(p0, p1, xw, indeg, b0, g0, be0):
    return pl.pallas_call(
        _tc_b1_body,
        grid=(NBLK,),
        in_specs=[_row_spec, _row_spec, _row_spec, _col_spec,
                  _vec128_spec, _vec128_spec, _vec128_spec],
        out_specs=[_row_spec,
                   pl.BlockSpec((1, 1, 1), lambda i: (i, 0, 0)),
                   pl.BlockSpec((1, 1, 1), lambda i: (i, 0, 0))],
        out_shape=[
            jax.ShapeDtypeStruct((NPAD, D_HID), jnp.float32),
            jax.ShapeDtypeStruct((NBLK, 1, 1), jnp.float32),
            jax.ShapeDtypeStruct((NBLK, 1, 1), jnp.float32),
        ],
    )(p0, p1, xw, indeg, b0, g0, be0)


def _tc_b2_body(h_ref, indeg_ref, sp_ref, sx_ref, w1_ref,
                y1_ref, hw1_ref, c1_ref, c2_ref):
    i = pl.program_id(0)
    h = h_ref[...]
    smin = jnp.min(sp_ref[...])
    smax = jnp.max(sx_ref[...])
    scores = jnp.sqrt(jnp.sum(h * h, axis=1, keepdims=True))
    scn = (scores - smin) / (smax - smin + 1e-8)
    rows = i * BLK + lax.broadcasted_iota(jnp.int32, (BLK, 1), 0)
    valid = rows < N
    m = jnp.where(jnp.logical_and(scn > LCS, valid), 1.0, 0.0)

    indeg = indeg_ref[...]
    dinv2 = lax.rsqrt(m * indeg + 1.0)
    hw1 = jnp.dot(h, w1_ref[...],
                  preferred_element_type=jnp.float32,
                  precision=lax.Precision.HIGHEST)
    y1 = hw1 * dinv2
    y1_ref[...] = jnp.concatenate(
        [y1, jnp.zeros((BLK, D_HID - D_OUT), jnp.float32)], axis=1)
    hw1_ref[...] = hw1
    c1_ref[...] = m * dinv2
    c2_ref[...] = dinv2 * dinv2


def _tc_b2(h, indeg, sp, sx, w1):
    return pl.pallas_call(
        _tc_b2_body,
        grid=(NBLK,),
        in_specs=[_row_spec, _col_spec,
                  pl.BlockSpec((NBLK, 1, 1), lambda i: (0, 0, 0)),
                  pl.BlockSpec((NBLK, 1, 1), lambda i: (0, 0, 0)),
                  pl.BlockSpec((D_HID, D_OUT), lambda i: (0, 0))],
        out_specs=[_row_spec,
                   pl.BlockSpec((BLK, D_OUT), lambda i: (i, 0)),
                   _col_spec, _col_spec],
        out_shape=[
            jax.ShapeDtypeStruct((NPAD, D_HID), jnp.float32),
            jax.ShapeDtypeStruct((NPAD, D_OUT), jnp.float32),
            jax.ShapeDtypeStruct((NPAD, 1), jnp.float32),
            jax.ShapeDtypeStruct((NPAD, 1), jnp.float32),
        ],
    )(h, indeg, sp, sx, w1)


def _tc_c_body(q0_ref, q1_ref, hw1_ref, c1_ref, c2_ref, b1_ref, out_ref):
    q = q0_ref[...] + q1_ref[...]
    o = (q[:, :D_OUT] * c1_ref[...]
         + c2_ref[...] * hw1_ref[...] + b1_ref[...][None, :])
    z = o - jnp.max(o, axis=1, keepdims=True)
    out_ref[...] = z - jnp.log(jnp.sum(jnp.exp(z), axis=1, keepdims=True))


def _tc_c(q0, q1, hw1, c1, c2, b1):
    return pl.pallas_call(
        _tc_c_body,
        grid=(NBLK,),
        in_specs=[_row_spec, _row_spec,
                  pl.BlockSpec((BLK, D_OUT), lambda i: (i, 0)),
                  _col_spec, _col_spec,
                  pl.BlockSpec((D_OUT,), lambda i: (0,))],
        out_specs=pl.BlockSpec((BLK, D_OUT), lambda i: (i, 0)),
        out_shape=jax.ShapeDtypeStruct((NPAD, D_OUT), jnp.float32),
    )(q0, q1, hw1, c1, c2, b1)


def kernel(x, edge_index, W0, b0, gamma0, beta0, W1, b1):
    src3 = edge_index[0].reshape(NW, NCHUNK, CHUNK)
    dst3 = edge_index[1].reshape(NW, NCHUNK, CHUNK)
    dst_flat = edge_index[1].reshape(NW, EPW)
    x_pad = jnp.pad(x, ((0, NPAD - N), (0, 0)))
    zeros_hid = jnp.zeros((CHUNK, D_HID), jnp.float32)

    hist = _sc_hist(dst_flat)
    xw, y0, indeg = _tc_a(hist, x_pad, W0)
    parts0 = _sc_segsum_hid(y0, src3, dst3, zeros_hid)
    h, sp, sx = _tc_b1(parts0[0], parts0[1], xw, indeg, b0, gamma0, beta0)
    y1, hw1, c1, c2 = _tc_b2(h, indeg, sp, sx, W1)
    parts1 = _sc_segsum_hid(y1, src3, dst3, zeros_hid)
    out = _tc_c(parts1[0], parts1[1], hw1, c1, c2, b1)
    return out[:N]


# speedup vs baseline: 31.5120x; 1.1293x over previous
import dataclasses
import functools

import jax
import jax.numpy as jnp
from jax import lax
from jax.experimental import pallas as pl
from jax.experimental.pallas import tpu as pltpu
from jax.experimental.pallas import tpu_sc as plsc

N = 10000
NPAD = 10240
E = 320000
D_IN = 128
D_HID = 128
D_OUT = 64
LCS = 0.1

NW = 32
EPW = E // NW
CHUNK = 128
NCHP = 79
EPP = NCHP * CHUNK
RPS = NPAD // 16

_MESH = dict(core_axis_name="c", subcore_axis_name="s")

_SC_CP = pltpu.CompilerParams()
if "needs_layout_passes" in pltpu.CompilerParams.__dataclass_fields__:
    _SC_CP = dataclasses.replace(_SC_CP, needs_layout_passes=False)


@functools.partial(
    pl.kernel,
    out_type=jax.ShapeDtypeStruct((NW, NPAD), jnp.float32),
    mesh=plsc.VectorSubcoreMesh(**_MESH),
    compiler_params=_SC_CP,
    scratch_types=[
        pltpu.VMEM((EPW,), jnp.int32),
        pltpu.VMEM((NPAD,), jnp.float32),
    ],
)
def _sc_hist(dst_hbm, out_hbm, dst_v, hist_v):
    wid = lax.axis_index("s") * 2 + lax.axis_index("c")
    pltpu.sync_copy(dst_hbm.at[wid], dst_v)

    zeros16 = jnp.zeros((16,), jnp.float32)

    @pl.loop(0, NPAD // 16)
    def _(i):
        hist_v[pl.ds(i * 16, 16)] = zeros16

    ones16 = jnp.ones((16,), jnp.float32)

    @pl.loop(0, EPW // 16)
    def _(j):
        idx = dst_v[pl.ds(j * 16, 16)]
        plsc.addupdate_scatter(hist_v, [idx], ones16)

    pltpu.sync_copy(hist_v, out_hbm.at[wid])


def _make_segsum(d):
    @functools.partial(
        pl.kernel,
        out_type=jax.ShapeDtypeStruct((2, NPAD, d), jnp.float32),
        mesh=plsc.VectorSubcoreMesh(**_MESH),
        scratch_types=[
            pltpu.VMEM((NCHP, CHUNK), jnp.int32),
            pltpu.VMEM((NCHP, CHUNK), jnp.int32),
            pltpu.VMEM((CHUNK, d), jnp.float32),
            pltpu.VMEM_SHARED((NPAD, d), jnp.float32),
        ],
    )
    def _sc_segsum(y_hbm, src_hbm, dst_hbm, zeros_hbm, out_hbm,
                   src_v, dst_v, rows_v, acc_s):
        core = lax.axis_index("c")
        sid = lax.axis_index("s")
        wid = sid * 2 + core
        base = sid * RPS

        pltpu.sync_copy(src_hbm.at[wid], src_v)
        pltpu.sync_copy(dst_hbm.at[wid], dst_v)

        pltpu.sync_copy(zeros_hbm, rows_v)

        @pl.loop(0, RPS // CHUNK)
        def _(k):
            pltpu.sync_copy(rows_v, acc_s.at[pl.ds(base + k * CHUNK, CHUNK)])

        plsc.subcore_barrier()

        @pl.loop(0, NCHP)
        def _(j):
            pltpu.sync_copy(y_hbm.at[src_v.at[j]], rows_v)
            pltpu.sync_copy(rows_v, acc_s.at[dst_v.at[j]], add=True)

        plsc.subcore_barrier()

        @pl.loop(0, RPS // CHUNK)
        def _(k):
            sl = pl.ds(base + k * CHUNK, CHUNK)
            pltpu.sync_copy(acc_s.at[sl], rows_v)
            pltpu.sync_copy(rows_v, out_hbm.at[core, sl])

    return _sc_segsum


_sc_segsum_hid = _make_segsum(D_HID)


BLK = 2048
NBLK = NPAD // BLK

_row_spec = pl.BlockSpec((BLK, D_HID), lambda i: (i, 0))
_col_spec = pl.BlockSpec((BLK, 1), lambda i: (i, 0))
_vec128_spec = pl.BlockSpec((D_HID,), lambda i: (0,))


def _tc_a_body(hist_ref, x_ref, w0_ref, xw_ref, y0_ref, indeg_ref):
    ones = jnp.ones((NW, 1), jnp.float32)
    indeg = lax.dot_general(hist_ref[...], ones, (((0,), (0,)), ((), ())),
                            preferred_element_type=jnp.float32)
    dinv = lax.rsqrt(indeg + 1.0)
    xw = jnp.dot(x_ref[...], w0_ref[...],
                 preferred_element_type=jnp.float32,
                 precision=lax.Precision.HIGHEST)
    xw_ref[...] = xw
    y0_ref[...] = xw * dinv
    indeg_ref[...] = indeg


def _tc_a(hist, x_pad, w0):
    return pl.pallas_call(
        _tc_a_body,
        grid=(NBLK,),
        in_specs=[
            pl.BlockSpec((NW, BLK), lambda i: (0, i)),
            _row_spec,
            pl.BlockSpec((D_IN, D_HID), lambda i: (0, 0)),
        ],
        out_specs=[_row_spec, _row_spec, _col_spec],
        out_shape=[
            jax.ShapeDtypeStruct((NPAD, D_HID), jnp.float32),
            jax.ShapeDtypeStruct((NPAD, D_HID), jnp.float32),
            jax.ShapeDtypeStruct((NPAD, 1), jnp.float32),
        ],
    )(hist, x_pad, w0)


def _tc_b1_body(p0_ref, p1_ref, xw_ref, indeg_ref, b0_ref, g0_ref, be0_ref,
                h_ref, smin_ref, smax_ref):
    i = pl.program_id(0)
    indeg = indeg_ref[...]
    dinv = lax.rsqrt(indeg + 1.0)
    agg = ((p0_ref[...] + p1_ref[...]) * dinv
           + dinv * dinv * xw_ref[...] + b0_ref[...][None, :])
    mu = jnp.mean(agg, axis=1, keepdims=True)
    cen = agg - mu
    var = jnp.mean(cen * cen, axis=1, keepdims=True)
    h = cen * lax.rsqrt(var + 1e-5) * g0_ref[...][None, :] + be0_ref[...][None, :]
    h = jnp.maximum(h, 0.0)
    h_ref[...] = h

    scores = jnp.sqrt(jnp.sum(h * h, axis=1, keepdims=True))
    rows = i * BLK + lax.broadcasted_iota(jnp.int32, (BLK, 1), 0)
    valid = rows < N
    big = jnp.float32(3e38)
    smin_ref[...] = jnp.min(jnp.where(valid, scores, big), keepdims=True)[None]
    smax_ref[...] = jnp.max(jnp.where(valid, scores, -big), keepdims=True)[None]


def _tc_b1(p0, p1, xw, indeg, b0, g0, be0):
    return pl.pallas_call(
        _tc_b1_body,
        grid=(NBLK,),
        in_specs=[_row_spec, _row_spec, _row_spec, _col_spec,
                  _vec128_spec, _vec128_spec, _vec128_spec],
        out_specs=[_row_spec,
                   pl.BlockSpec((1, 1, 1), lambda i: (i, 0, 0)),
                   pl.BlockSpec((1, 1, 1), lambda i: (i, 0, 0))],
        out_shape=[
            jax.ShapeDtypeStruct((NPAD, D_HID), jnp.float32),
            jax.ShapeDtypeStruct((NBLK, 1, 1), jnp.float32),
            jax.ShapeDtypeStruct((NBLK, 1, 1), jnp.float32),
        ],
    )(p0, p1, xw, indeg, b0, g0, be0)


def _tc_b2_body(h_ref, indeg_ref, sp_ref, sx_ref, w1_ref,
                y1_ref, hw1_ref, c1_ref, c2_ref):
    i = pl.program_id(0)
    h = h_ref[...]
    smin = jnp.min(sp_ref[...])
    smax = jnp.max(sx_ref[...])
    scores = jnp.sqrt(jnp.sum(h * h, axis=1, keepdims=True))
    scn = (scores - smin) / (smax - smin + 1e-8)
    rows = i * BLK + lax.broadcasted_iota(jnp.int32, (BLK, 1), 0)
    valid = rows < N
    m = jnp.where(jnp.logical_and(scn > LCS, valid), 1.0, 0.0)

    indeg = indeg_ref[...]
    dinv2 = lax.rsqrt(m * indeg + 1.0)
    hw1 = jnp.dot(h, w1_ref[...],
                  preferred_element_type=jnp.float32,
                  precision=lax.Precision.HIGHEST)
    y1 = hw1 * dinv2
    y1_ref[...] = jnp.concatenate(
        [y1, jnp.zeros((BLK, D_HID - D_OUT), jnp.float32)], axis=1)
    hw1_ref[...] = hw1
    c1_ref[...] = m * dinv2
    c2_ref[...] = dinv2 * dinv2


def _tc_b2(h, indeg, sp, sx, w1):
    return pl.pallas_call(
        _tc_b2_body,
        grid=(NBLK,),
        in_specs=[_row_spec, _col_spec,
                  pl.BlockSpec((NBLK, 1, 1), lambda i: (0, 0, 0)),
                  pl.BlockSpec((NBLK, 1, 1), lambda i: (0, 0, 0)),
                  pl.BlockSpec((D_HID, D_OUT), lambda i: (0, 0))],
        out_specs=[_row_spec,
                   pl.BlockSpec((BLK, D_OUT), lambda i: (i, 0)),
                   _col_spec, _col_spec],
        out_shape=[
            jax.ShapeDtypeStruct((NPAD, D_HID), jnp.float32),
            jax.ShapeDtypeStruct((NPAD, D_OUT), jnp.float32),
            jax.ShapeDtypeStruct((NPAD, 1), jnp.float32),
            jax.ShapeDtypeStruct((NPAD, 1), jnp.float32),
        ],
    )(h, indeg, sp, sx, w1)


def _tc_c_body(q0_ref, q1_ref, hw1_ref, c1_ref, c2_ref, b1_ref, out_ref):
    q = q0_ref[...] + q1_ref[...]
    o = (q[:, :D_OUT] * c1_ref[...]
         + c2_ref[...] * hw1_ref[...] + b1_ref[...][None, :])
    z = o - jnp.max(o, axis=1, keepdims=True)
    out_ref[...] = z - jnp.log(jnp.sum(jnp.exp(z), axis=1, keepdims=True))


def _tc_c(q0, q1, hw1, c1, c2, b1):
    return pl.pallas_call(
        _tc_c_body,
        grid=(NBLK,),
        in_specs=[_row_spec, _row_spec,
                  pl.BlockSpec((BLK, D_OUT), lambda i: (i, 0)),
                  _col_spec, _col_spec,
                  pl.BlockSpec((D_OUT,), lambda i: (0,))],
        out_specs=pl.BlockSpec((BLK, D_OUT), lambda i: (i, 0)),
        out_shape=jax.ShapeDtypeStruct((NPAD, D_OUT), jnp.float32),
    )(q0, q1, hw1, c1, c2, b1)


def kernel(x, edge_index, W0, b0, gamma0, beta0, W1, b1):
    ndum = EPP - EPW
    dum = jnp.arange(ndum, dtype=jnp.int32)
    dum_src = jnp.broadcast_to((dum * 97) % N, (NW, ndum))
    dum_dst = jnp.broadcast_to(N + dum % (NPAD - N), (NW, ndum))
    src3 = jnp.concatenate(
        [edge_index[0].reshape(NW, EPW), dum_src], axis=1
    ).reshape(NW, NCHP, CHUNK)
    dst3 = jnp.concatenate(
        [edge_index[1].reshape(NW, EPW), dum_dst], axis=1
    ).reshape(NW, NCHP, CHUNK)
    dst_flat = edge_index[1].reshape(NW, EPW)
    x_pad = jnp.pad(x, ((0, NPAD - N), (0, 0)))
    zeros_hid = jnp.zeros((CHUNK, D_HID), jnp.float32)

    hist = _sc_hist(dst_flat)
    xw, y0, indeg = _tc_a(hist, x_pad, W0)
    parts0 = _sc_segsum_hid(y0, src3, dst3, zeros_hid)
    h, sp, sx = _tc_b1(parts0[0], parts0[1], xw, indeg, b0, gamma0, beta0)
    y1, hw1, c1, c2 = _tc_b2(h, indeg, sp, sx, W1)
    parts1 = _sc_segsum_hid(y1, src3, dst3, zeros_hid)
    out = _tc_c(parts1[0], parts1[1], hw1, c1, c2, b1)
    return out[:N]


# speedup vs baseline: 31.5985x; 1.0027x over previous
import dataclasses
import functools

import jax
import jax.numpy as jnp
from jax import lax
from jax.experimental import pallas as pl
from jax.experimental.pallas import tpu as pltpu
from jax.experimental.pallas import tpu_sc as plsc

N = 10000
NPAD = 10240
E = 320000
D_IN = 128
D_HID = 128
D_OUT = 64
LCS = 0.1

NW = 32
EPW = E // NW
CHUNK = 64
UROW = 128
NROW = 80
NUNIT = 160
EPP = NROW * UROW
WCH = 128
RPS = NPAD // 16

_MESH = dict(core_axis_name="c", subcore_axis_name="s")

_SC_CP = pltpu.CompilerParams()
if "needs_layout_passes" in pltpu.CompilerParams.__dataclass_fields__:
    _SC_CP = dataclasses.replace(_SC_CP, needs_layout_passes=False)


@functools.partial(
    pl.kernel,
    out_type=jax.ShapeDtypeStruct((NW, NPAD), jnp.float32),
    mesh=plsc.VectorSubcoreMesh(**_MESH),
    compiler_params=_SC_CP,
    scratch_types=[
        pltpu.VMEM((EPW,), jnp.int32),
        pltpu.VMEM((NPAD,), jnp.float32),
    ],
)
def _sc_hist(dst_hbm, out_hbm, dst_v, hist_v):
    wid = lax.axis_index("s") * 2 + lax.axis_index("c")
    pltpu.sync_copy(dst_hbm.at[wid], dst_v)

    zeros16 = jnp.zeros((16,), jnp.float32)

    @pl.loop(0, NPAD // 16)
    def _(i):
        hist_v[pl.ds(i * 16, 16)] = zeros16

    ones16 = jnp.ones((16,), jnp.float32)

    @pl.loop(0, EPW // 16)
    def _(j):
        idx = dst_v[pl.ds(j * 16, 16)]
        plsc.addupdate_scatter(hist_v, [idx], ones16)

    pltpu.sync_copy(hist_v, out_hbm.at[wid])


def _make_segsum(d):
    @functools.partial(
        pl.kernel,
        out_type=jax.ShapeDtypeStruct((2, NPAD, d), jnp.float32),
        mesh=plsc.VectorSubcoreMesh(**_MESH),
        scratch_types=[
            pltpu.VMEM((2, NROW, UROW), jnp.int32),
            pltpu.VMEM((2 * CHUNK, d), jnp.float32),
            pltpu.VMEM_SHARED((NPAD, d), jnp.float32),
            pltpu.SemaphoreType.DMA,
        ],
    )
    def _sc_segsum(y_hbm, edges_hbm, out_hbm, idx_v, buf_v, acc_s, sem):
        core = lax.axis_index("c")
        sid = lax.axis_index("s")
        wid = sid * 2 + core
        base = sid * RPS

        pltpu.sync_copy(edges_hbm.at[wid], idx_v)

        zeros16 = jnp.zeros((16,), jnp.float32)

        @pl.loop(0, WCH)
        def _(r):
            @pl.loop(0, d // 16)
            def _(c):
                buf_v[r, pl.ds(c * 16, 16)] = zeros16

        @pl.loop(0, RPS // WCH)
        def _(k):
            pltpu.sync_copy(buf_v, acc_s.at[pl.ds(base + k * WCH, WCH)])

        plsc.subcore_barrier()

        @pl.loop(0, NUNIT)
        def _(j):
            row = lax.div(j, 2)
            s = lax.rem(j, 2) * CHUNK
            p = CHUNK - s
            src_sl = idx_v.at[0, row, pl.ds(s, CHUNK)]
            dst_sl = idx_v.at[1, row, pl.ds(s, CHUNK)]
            pltpu.async_copy(y_hbm.at[src_sl], buf_v.at[pl.ds(s, CHUNK)],
                             sem)
            pltpu.sync_copy(buf_v.at[pl.ds(p, CHUNK)],
                            acc_s.at[dst_sl], add=True)
            pltpu.make_async_copy(y_hbm.at[src_sl],
                                  buf_v.at[pl.ds(s, CHUNK)], sem).wait()

        plsc.subcore_barrier()

        @pl.loop(0, RPS // WCH)
        def _(k):
            sl = pl.ds(base + k * WCH, WCH)
            pltpu.sync_copy(acc_s.at[sl], buf_v)
            pltpu.sync_copy(buf_v, out_hbm.at[core, sl])

    return _sc_segsum


_sc_segsum_hid = _make_segsum(D_HID)


BLK = 2048
NBLK = NPAD // BLK

_row_spec = pl.BlockSpec((BLK, D_HID), lambda i: (i, 0))
_col_spec = pl.BlockSpec((BLK, 1), lambda i: (i, 0))
_vec128_spec = pl.BlockSpec((D_HID,), lambda i: (0,))


def _tc_a_body(hist_ref, x_ref, w0_ref, xw_ref, y0_ref, indeg_ref):
    ones = jnp.ones((NW, 1), jnp.float32)
    indeg = lax.dot_general(hist_ref[...], ones, (((0,), (0,)), ((), ())),
                            preferred_element_type=jnp.float32)
    dinv = lax.rsqrt(indeg + 1.0)
    xw = jnp.dot(x_ref[...], w0_ref[...],
                 preferred_element_type=jnp.float32,
                 precision=lax.Precision.HIGHEST)
    xw_ref[...] = xw
    y0_ref[...] = xw * dinv
    indeg_ref[...] = indeg


def _tc_a(hist, x_pad, w0):
    return pl.pallas_call(
        _tc_a_body,
        grid=(NBLK,),
        in_specs=[
            pl.BlockSpec((NW, BLK), lambda i: (0, i)),
            _row_spec,
            pl.BlockSpec((D_IN, D_HID), lambda i: (0, 0)),
        ],
        out_specs=[_row_spec, _row_spec, _col_spec],
        out_shape=[
            jax.ShapeDtypeStruct((NPAD, D_HID), jnp.float32),
            jax.ShapeDtypeStruct((NPAD, D_HID), jnp.float32),
            jax.ShapeDtypeStruct((NPAD, 1), jnp.float32),
        ],
    )(hist, x_pad, w0)


def _tc_b1_body(p0_ref, p1_ref, xw_ref, indeg_ref, b0_ref, g0_ref, be0_ref,
                h_ref, smin_ref, smax_ref):
    i = pl.program_id(0)
    indeg = indeg_ref[...]
    dinv = lax.rsqrt(indeg + 1.0)
    agg = ((p0_ref[...] + p1_ref[...]) * dinv
           + dinv * dinv * xw_ref[...] + b0_ref[...][None, :])
    mu = jnp.mean(agg, axis=1, keepdims=True)
    cen = agg - mu
    var = jnp.mean(cen * cen, axis=1, keepdims=True)
    h = cen * lax.rsqrt(var + 1e-5) * g0_ref[...][None, :] + be0_ref[...][None, :]
    h = jnp.maximum(h, 0.0)
    h_ref[...] = h

    scores = jnp.sqrt(jnp.sum(h * h, axis=1, keepdims=True))
    rows = i * BLK + lax.broadcasted_iota(jnp.int32, (BLK, 1), 0)
    valid = rows < N
    big = jnp.float32(3e38)
    smin_ref[...] = jnp.min(jnp.where(valid, scores, big), keepdims=True)[None]
    smax_ref[...] = jnp.max(jnp.where(valid, scores, -big), keepdims=True)[None]


def _tc_b1(p0, p1, xw, indeg, b0, g0, be0):
    return pl.pallas_call(
        _tc_b1_body,
        grid=(NBLK,),
        in_specs=[_row_spec, _row_spec, _row_spec, _col_spec,
                  _vec128_spec, _vec128_spec, _vec128_spec],
        out_specs=[_row_spec,
                   pl.BlockSpec((1, 1, 1), lambda i: (i, 0, 0)),
                   pl.BlockSpec((1, 1, 1), lambda i: (i, 0, 0))],
        out_shape=[
            jax.ShapeDtypeStruct((NPAD, D_HID), jnp.float32),
            jax.ShapeDtypeStruct((NBLK, 1, 1), jnp.float32),
            jax.ShapeDtypeStruct((NBLK, 1, 1), jnp.float32),
        ],
    )(p0, p1, xw, indeg, b0, g0, be0)


def _tc_b2_body(h_ref, indeg_ref, sp_ref, sx_ref, w1_ref,
                y1_ref, hw1_ref, c1_ref, c2_ref):
    i = pl.program_id(0)
    h = h_ref[...]
    smin = jnp.min(sp_ref[...])
    smax = jnp.max(sx_ref[...])
    scores = jnp.sqrt(jnp.sum(h * h, axis=1, keepdims=True))
    scn = (scores - smin) / (smax - smin + 1e-8)
    rows = i * BLK + lax.broadcasted_iota(jnp.int32, (BLK, 1), 0)
    valid = rows < N
    m = jnp.where(jnp.logical_and(scn > LCS, valid), 1.0, 0.0)

    indeg = indeg_ref[...]
    dinv2 = lax.rsqrt(m * indeg + 1.0)
    hw1 = jnp.dot(h, w1_ref[...],
                  preferred_element_type=jnp.float32,
                  precision=lax.Precision.HIGHEST)
    y1 = hw1 * dinv2
    y1_ref[...] = jnp.concatenate(
        [y1, jnp.zeros((BLK, D_HID - D_OUT), jnp.float32)], axis=1)
    hw1_ref[...] = hw1
    c1_ref[...] = m * dinv2
    c2_ref[...] = dinv2 * dinv2


def _tc_b2(h, indeg, sp, sx, w1):
    return pl.pallas_call(
        _tc_b2_body,
        grid=(NBLK,),
        in_specs=[_row_spec, _col_spec,
                  pl.BlockSpec((NBLK, 1, 1), lambda i: (0, 0, 0)),
                  pl.BlockSpec((NBLK, 1, 1), lambda i: (0, 0, 0)),
                  pl.BlockSpec((D_HID, D_OUT), lambda i: (0, 0))],
        out_specs=[_row_spec,
                   pl.BlockSpec((BLK, D_OUT), lambda i: (i, 0)),
                   _col_spec, _col_spec],
        out_shape=[
            jax.ShapeDtypeStruct((NPAD, D_HID), jnp.float32),
            jax.ShapeDtypeStruct((NPAD, D_OUT), jnp.float32),
            jax.ShapeDtypeStruct((NPAD, 1), jnp.float32),
            jax.ShapeDtypeStruct((NPAD, 1), jnp.float32),
        ],
    )(h, indeg, sp, sx, w1)


def _tc_c_body(q0_ref, q1_ref, hw1_ref, c1_ref, c2_ref, b1_ref, out_ref):
    q = q0_ref[...] + q1_ref[...]
    o = (q[:, :D_OUT] * c1_ref[...]
         + c2_ref[...] * hw1_ref[...] + b1_ref[...][None, :])
    z = o - jnp.max(o, axis=1, keepdims=True)
    out_ref[...] = z - jnp.log(jnp.sum(jnp.exp(z), axis=1, keepdims=True))


def _tc_c(q0, q1, hw1, c1, c2, b1):
    return pl.pallas_call(
        _tc_c_body,
        grid=(NBLK,),
        in_specs=[_row_spec, _row_spec,
                  pl.BlockSpec((BLK, D_OUT), lambda i: (i, 0)),
                  _col_spec, _col_spec,
                  pl.BlockSpec((D_OUT,), lambda i: (0,))],
        out_specs=pl.BlockSpec((BLK, D_OUT), lambda i: (i, 0)),
        out_shape=jax.ShapeDtypeStruct((NPAD, D_OUT), jnp.float32),
    )(q0, q1, hw1, c1, c2, b1)


def kernel(x, edge_index, W0, b0, gamma0, beta0, W1, b1):
    nds = EPP - EPW
    ds_ = jnp.arange(nds, dtype=jnp.int32)
    dum_src = jnp.broadcast_to((ds_ * 97) % N, (NW, nds))
    ndd = EPP - EPW - CHUNK
    dd_ = jnp.arange(ndd, dtype=jnp.int32)
    dum_dst = jnp.broadcast_to(N + dd_ % (NPAD - N), (NW, ndd))
    ddl = jnp.arange(CHUNK, dtype=jnp.int32)
    dum_dst0 = jnp.broadcast_to(N + ddl % (NPAD - N), (NW, CHUNK))
    srcA = jnp.concatenate(
        [edge_index[0].reshape(NW, EPW), dum_src], axis=1
    ).reshape(NW, 1, NROW, UROW)
    dstA = jnp.concatenate(
        [dum_dst0, edge_index[1].reshape(NW, EPW), dum_dst], axis=1
    ).reshape(NW, 1, NROW, UROW)
    edges = jnp.concatenate([srcA, dstA], axis=1)
    dst_flat = edge_index[1].reshape(NW, EPW)
    x_pad = jnp.pad(x, ((0, NPAD - N), (0, 0)))

    hist = _sc_hist(dst_flat)
    xw, y0, indeg = _tc_a(hist, x_pad, W0)
    parts0 = _sc_segsum_hid(y0, edges)
    h, sp, sx = _tc_b1(parts0[0], parts0[1], xw, indeg, b0, gamma0, beta0)
    y1, hw1, c1, c2 = _tc_b2(h, indeg, sp, sx, W1)
    parts1 = _sc_segsum_hid(y1, edges)
    out = _tc_c(parts1[0], parts1[1], hw1, c1, c2, b1)
    return out[:N]


# speedup vs baseline: 31.8278x; 1.0073x over previous
import dataclasses
import functools

import jax
import jax.numpy as jnp
from jax import lax
from jax.experimental import pallas as pl
from jax.experimental.pallas import tpu as pltpu
from jax.experimental.pallas import tpu_sc as plsc

N = 10000
NPAD = 10240
E = 320000
D_IN = 128
D_HID = 128
D_OUT = 64
LCS = 0.1

NW = 32
EPW = E // NW
CHUNK = 64
UROW = 128
NROW = 80
NUNIT = 160
EPP = NROW * UROW
WCH = 128
RPS = NPAD // 16

_MESH = dict(core_axis_name="c", subcore_axis_name="s")

_SC_CP = pltpu.CompilerParams()
if "needs_layout_passes" in pltpu.CompilerParams.__dataclass_fields__:
    _SC_CP = dataclasses.replace(_SC_CP, needs_layout_passes=False)


@functools.partial(
    pl.kernel,
    out_type=jax.ShapeDtypeStruct((NW, NPAD), jnp.float32),
    mesh=plsc.VectorSubcoreMesh(**_MESH),
    compiler_params=_SC_CP,
    scratch_types=[
        pltpu.VMEM((EPW,), jnp.int32),
        pltpu.VMEM((NPAD,), jnp.float32),
    ],
)
def _sc_hist(dst_hbm, out_hbm, dst_v, hist_v):
    wid = lax.axis_index("s") * 2 + lax.axis_index("c")
    pltpu.sync_copy(dst_hbm.at[wid], dst_v)

    zeros16 = jnp.zeros((16,), jnp.float32)

    @pl.loop(0, NPAD // 16)
    def _(i):
        hist_v[pl.ds(i * 16, 16)] = zeros16

    ones16 = jnp.ones((16,), jnp.float32)

    @pl.loop(0, EPW // 16)
    def _(j):
        idx = dst_v[pl.ds(j * 16, 16)]
        plsc.addupdate_scatter(hist_v, [idx], ones16)

    pltpu.sync_copy(hist_v, out_hbm.at[wid])


def _make_segsum(d):
    @functools.partial(
        pl.kernel,
        out_type=jax.ShapeDtypeStruct((2, NPAD, d), jnp.float32),
        mesh=plsc.VectorSubcoreMesh(**_MESH),
        scratch_types=[
            pltpu.VMEM((2, NROW, UROW), jnp.int32),
            pltpu.VMEM((2 * CHUNK, d), jnp.float32),
            pltpu.VMEM_SHARED((NPAD, d), jnp.float32),
            pltpu.SemaphoreType.DMA,
        ],
    )
    def _sc_segsum(y_hbm, edges_hbm, out_hbm, idx_v, buf_v, acc_s, sem):
        core = lax.axis_index("c")
        sid = lax.axis_index("s")
        wid = sid * 2 + core
        base = sid * RPS

        pltpu.sync_copy(edges_hbm.at[wid], idx_v)

        zeros16 = jnp.zeros((16,), jnp.float32)

        @pl.loop(0, WCH)
        def _(r):
            @pl.loop(0, d // 16)
            def _(c):
                buf_v[r, pl.ds(c * 16, 16)] = zeros16

        @pl.loop(0, RPS // WCH)
        def _(k):
            pltpu.sync_copy(buf_v, acc_s.at[pl.ds(base + k * WCH, WCH)])

        plsc.subcore_barrier()

        @pl.loop(0, NUNIT)
        def _(j):
            row = lax.div(j, 2)
            s = lax.rem(j, 2) * CHUNK
            p = CHUNK - s
            src_sl = idx_v.at[0, row, pl.ds(s, CHUNK)]
            dst_sl = idx_v.at[1, row, pl.ds(s, CHUNK)]
            pltpu.async_copy(y_hbm.at[src_sl], buf_v.at[pl.ds(s, CHUNK)],
                             sem)
            pltpu.sync_copy(buf_v.at[pl.ds(p, CHUNK)],
                            acc_s.at[dst_sl], add=True)
            pltpu.make_async_copy(y_hbm.at[src_sl],
                                  buf_v.at[pl.ds(s, CHUNK)], sem).wait()

        plsc.subcore_barrier()

        @pl.loop(0, RPS // WCH)
        def _(k):
            sl = pl.ds(base + k * WCH, WCH)
            pltpu.sync_copy(acc_s.at[sl], buf_v)
            pltpu.sync_copy(buf_v, out_hbm.at[core, sl])

    return _sc_segsum


_sc_segsum_hid = _make_segsum(D_HID)


BLK = 2048
NBLK = NPAD // BLK

_row_spec = pl.BlockSpec((BLK, D_HID), lambda i: (i, 0))
_col_spec = pl.BlockSpec((BLK, 1), lambda i: (i, 0))
_vec128_spec = pl.BlockSpec((D_HID,), lambda i: (0,))


def _tc_a_body(hist_ref, x_ref, w0_ref, xw_ref, y0_ref, indeg_ref):
    ones = jnp.ones((NW, 1), jnp.float32)
    indeg = lax.dot_general(hist_ref[...], ones, (((0,), (0,)), ((), ())),
                            preferred_element_type=jnp.float32)
    dinv = lax.rsqrt(indeg + 1.0)
    xw = jnp.dot(x_ref[...], w0_ref[...],
                 preferred_element_type=jnp.float32,
                 precision=lax.Precision.HIGHEST)
    xw_ref[...] = xw
    y0_ref[...] = xw * dinv
    indeg_ref[...] = indeg


def _tc_a(hist, x_pad, w0):
    return pl.pallas_call(
        _tc_a_body,
        grid=(NBLK,),
        in_specs=[
            pl.BlockSpec((NW, BLK), lambda i: (0, i)),
            _row_spec,
            pl.BlockSpec((D_IN, D_HID), lambda i: (0, 0)),
        ],
        out_specs=[_row_spec, _row_spec, _col_spec],
        out_shape=[
            jax.ShapeDtypeStruct((NPAD, D_HID), jnp.float32),
            jax.ShapeDtypeStruct((NPAD, D_HID), jnp.float32),
            jax.ShapeDtypeStruct((NPAD, 1), jnp.float32),
        ],
    )(hist, x_pad, w0)


def _tc_b_body(p_ref, xw_ref, indeg_ref, b0_ref, g0_ref, be0_ref, w1_ref,
               y1_ref, hw1_ref, c1_ref, c2_ref, h_s, mm_s):
    ph = pl.program_id(0)
    i = pl.program_id(1)
    rows = i * BLK + lax.broadcasted_iota(jnp.int32, (BLK, 1), 0)
    valid = rows < N
    indeg = indeg_ref[...]

    @pl.when(ph == 0)
    def _():
        dinv = lax.rsqrt(indeg + 1.0)
        agg = ((p_ref[0] + p_ref[1]) * dinv
               + dinv * dinv * xw_ref[...] + b0_ref[...][None, :])
        mu = jnp.mean(agg, axis=1, keepdims=True)
        cen = agg - mu
        var = jnp.mean(cen * cen, axis=1, keepdims=True)
        h = (cen * lax.rsqrt(var + 1e-5) * g0_ref[...][None, :]
             + be0_ref[...][None, :])
        h = jnp.maximum(h, 0.0)
        h_s[pl.ds(i * BLK, BLK), :] = h
        scores = jnp.sqrt(jnp.sum(h * h, axis=1, keepdims=True))
        big = jnp.float32(3e38)
        mm_s[0, i] = jnp.min(jnp.where(valid, scores, big))
        mm_s[1, i] = jnp.max(jnp.where(valid, scores, -big))
        y1_ref[...] = jnp.zeros((BLK, D_HID), jnp.float32)
        hw1_ref[...] = jnp.zeros((BLK, D_OUT), jnp.float32)
        c1_ref[...] = jnp.zeros((BLK, 1), jnp.float32)
        c2_ref[...] = jnp.zeros((BLK, 1), jnp.float32)

    @pl.when(ph == 1)
    def _():
        smin = mm_s[0, 0]
        smax = mm_s[1, 0]
        for k in range(1, NBLK):
            smin = jnp.minimum(smin, mm_s[0, k])
            smax = jnp.maximum(smax, mm_s[1, k])
        h = h_s[pl.ds(i * BLK, BLK), :]
        scores = jnp.sqrt(jnp.sum(h * h, axis=1, keepdims=True))
        scn = (scores - smin) / (smax - smin + 1e-8)
        m = jnp.where(jnp.logical_and(scn > LCS, valid), 1.0, 0.0)
        dinv2 = lax.rsqrt(m * indeg + 1.0)
        hw1 = jnp.dot(h, w1_ref[...],
                      preferred_element_type=jnp.float32,
                      precision=lax.Precision.HIGHEST)
        y1 = hw1 * dinv2
        y1_ref[...] = jnp.concatenate(
            [y1, jnp.zeros((BLK, D_HID - D_OUT), jnp.float32)], axis=1)
        hw1_ref[...] = hw1
        c1_ref[...] = m * dinv2
        c2_ref[...] = dinv2 * dinv2


def _tc_b(p, xw, indeg, b0, g0, be0, w1):
    return pl.pallas_call(
        _tc_b_body,
        grid=(2, NBLK),
        in_specs=[pl.BlockSpec((2, BLK, D_HID), lambda ph, i: (0, i, 0)),
                  pl.BlockSpec((BLK, D_HID), lambda ph, i: (i, 0)),
                  pl.BlockSpec((BLK, 1), lambda ph, i: (i, 0)),
                  pl.BlockSpec((D_HID,), lambda ph, i: (0,)),
                  pl.BlockSpec((D_HID,), lambda ph, i: (0,)),
                  pl.BlockSpec((D_HID,), lambda ph, i: (0,)),
                  pl.BlockSpec((D_HID, D_OUT), lambda ph, i: (0, 0))],
        out_specs=[pl.BlockSpec((BLK, D_HID), lambda ph, i: (i, 0)),
                   pl.BlockSpec((BLK, D_OUT), lambda ph, i: (i, 0)),
                   pl.BlockSpec((BLK, 1), lambda ph, i: (i, 0)),
                   pl.BlockSpec((BLK, 1), lambda ph, i: (i, 0))],
        out_shape=[
            jax.ShapeDtypeStruct((NPAD, D_HID), jnp.float32),
            jax.ShapeDtypeStruct((NPAD, D_OUT), jnp.float32),
            jax.ShapeDtypeStruct((NPAD, 1), jnp.float32),
            jax.ShapeDtypeStruct((NPAD, 1), jnp.float32),
        ],
        scratch_shapes=[
            pltpu.VMEM((NPAD, D_HID), jnp.float32),
            pltpu.SMEM((2, NBLK), jnp.float32),
        ],
    )(p, xw, indeg, b0, g0, be0, w1)


def _tc_c_body(q0_ref, q1_ref, hw1_ref, c1_ref, c2_ref, b1_ref, out_ref):
    q = q0_ref[...] + q1_ref[...]
    o = (q[:, :D_OUT] * c1_ref[...]
         + c2_ref[...] * hw1_ref[...] + b1_ref[...][None, :])
    z = o - jnp.max(o, axis=1, keepdims=True)
    out_ref[...] = z - jnp.log(jnp.sum(jnp.exp(z), axis=1, keepdims=True))


def _tc_c(q0, q1, hw1, c1, c2, b1):
    return pl.pallas_call(
        _tc_c_body,
        grid=(NBLK,),
        in_specs=[_row_spec, _row_spec,
                  pl.BlockSpec((BLK, D_OUT), lambda i: (i, 0)),
                  _col_spec, _col_spec,
                  pl.BlockSpec((D_OUT,), lambda i: (0,))],
        out_specs=pl.BlockSpec((BLK, D_OUT), lambda i: (i, 0)),
        out_shape=jax.ShapeDtypeStruct((NPAD, D_OUT), jnp.float32),
    )(q0, q1, hw1, c1, c2, b1)


def kernel(x, edge_index, W0, b0, gamma0, beta0, W1, b1):
    nds = EPP - EPW
    ds_ = jnp.arange(nds, dtype=jnp.int32)
    dum_src = jnp.broadcast_to((ds_ * 97) % N, (NW, nds))
    ndd = EPP - EPW - CHUNK
    dd_ = jnp.arange(ndd, dtype=jnp.int32)
    dum_dst = jnp.broadcast_to(N + dd_ % (NPAD - N), (NW, ndd))
    ddl = jnp.arange(CHUNK, dtype=jnp.int32)
    dum_dst0 = jnp.broadcast_to(N + ddl % (NPAD - N), (NW, CHUNK))
    srcA = jnp.concatenate(
        [edge_index[0].reshape(NW, EPW), dum_src], axis=1
    ).reshape(NW, 1, NROW, UROW)
    dstA = jnp.concatenate(
        [dum_dst0, edge_index[1].reshape(NW, EPW), dum_dst], axis=1
    ).reshape(NW, 1, NROW, UROW)
    edges = jnp.concatenate([srcA, dstA], axis=1)
    dst_flat = edge_index[1].reshape(NW, EPW)
    x_pad = jnp.pad(x, ((0, NPAD - N), (0, 0)))

    hist = _sc_hist(dst_flat)
    xw, y0, indeg = _tc_a(hist, x_pad, W0)
    parts0 = _sc_segsum_hid(y0, edges)
    y1, hw1, c1, c2 = _tc_b(parts0, xw, indeg, b0, gamma0, beta0, W1)
    parts1 = _sc_segsum_hid(y1, edges)
    out = _tc_c(parts1[0], parts1[1], hw1, c1, c2, b1)
    return out[:N]


# speedup vs baseline: 31.9492x; 1.0038x over previous
import dataclasses
import functools

import jax
import jax.numpy as jnp
from jax import lax
from jax.experimental import pallas as pl
from jax.experimental.pallas import tpu as pltpu
from jax.experimental.pallas import tpu_sc as plsc

N = 10000
NPAD = 10240
E = 320000
D_IN = 128
D_HID = 128
D_OUT = 64
LCS = 0.1

NW = 32
EPW = E // NW
CHUNK = 64
UROW = 128
NROW = 80
NUNIT = 160
EPP = NROW * UROW
WCH = 128
RPS = NPAD // 16

_MESH = dict(core_axis_name="c", subcore_axis_name="s")

_SC_CP = pltpu.CompilerParams()
if "needs_layout_passes" in pltpu.CompilerParams.__dataclass_fields__:
    _SC_CP = dataclasses.replace(_SC_CP, needs_layout_passes=False)


@functools.partial(
    pl.kernel,
    out_type=jax.ShapeDtypeStruct((NW, NPAD), jnp.float32),
    mesh=plsc.VectorSubcoreMesh(**_MESH),
    compiler_params=_SC_CP,
    scratch_types=[
        pltpu.VMEM((EPW,), jnp.int32),
        pltpu.VMEM((NPAD,), jnp.float32),
    ],
)
def _sc_hist(dst_hbm, out_hbm, dst_v, hist_v):
    wid = lax.axis_index("s") * 2 + lax.axis_index("c")
    pltpu.sync_copy(dst_hbm.at[wid], dst_v)

    zeros16 = jnp.zeros((16,), jnp.float32)

    @pl.loop(0, NPAD // 16)
    def _(i):
        hist_v[pl.ds(i * 16, 16)] = zeros16

    ones16 = jnp.ones((16,), jnp.float32)

    @pl.loop(0, EPW // 16)
    def _(j):
        idx = dst_v[pl.ds(j * 16, 16)]
        plsc.addupdate_scatter(hist_v, [idx], ones16)

    pltpu.sync_copy(hist_v, out_hbm.at[wid])


def _make_segsum(d):
    @functools.partial(
        pl.kernel,
        out_type=jax.ShapeDtypeStruct((2, NPAD, d), jnp.float32),
        mesh=plsc.VectorSubcoreMesh(**_MESH),
        scratch_types=[
            pltpu.VMEM((2, NROW, UROW), jnp.int32),
            pltpu.VMEM((2 * CHUNK, d), jnp.float32),
            pltpu.VMEM_SHARED((NPAD, d), jnp.float32),
            pltpu.SemaphoreType.DMA,
        ],
    )
    def _sc_segsum(y_hbm, edges_hbm, out_hbm, idx_v, buf_v, acc_s, sem):
        core = lax.axis_index("c")
        sid = lax.axis_index("s")
        wid = sid * 2 + core
        base = sid * RPS

        pltpu.sync_copy(edges_hbm.at[wid], idx_v)

        zeros16 = jnp.zeros((16,), jnp.float32)

        @pl.loop(0, WCH)
        def _(r):
            @pl.loop(0, d // 16)
            def _(c):
                buf_v[r, pl.ds(c * 16, 16)] = zeros16

        @pl.loop(0, RPS // WCH)
        def _(k):
            pltpu.sync_copy(buf_v, acc_s.at[pl.ds(base + k * WCH, WCH)])

        plsc.subcore_barrier()

        @pl.loop(0, NUNIT)
        def _(j):
            row = lax.div(j, 2)
            s = lax.rem(j, 2) * CHUNK
            p = CHUNK - s
            src_sl = idx_v.at[0, row, pl.ds(s, CHUNK)]
            dst_sl = idx_v.at[1, row, pl.ds(s, CHUNK)]
            pltpu.async_copy(y_hbm.at[src_sl], buf_v.at[pl.ds(s, CHUNK)],
                             sem)
            pltpu.sync_copy(buf_v.at[pl.ds(p, CHUNK)],
                            acc_s.at[dst_sl], add=True)
            pltpu.make_async_copy(y_hbm.at[src_sl],
                                  buf_v.at[pl.ds(s, CHUNK)], sem).wait()

        plsc.subcore_barrier()

        @pl.loop(0, RPS // WCH)
        def _(k):
            sl = pl.ds(base + k * WCH, WCH)
            pltpu.sync_copy(acc_s.at[sl], buf_v)
            pltpu.sync_copy(buf_v, out_hbm.at[core, sl])

    return _sc_segsum


_sc_segsum_hid = _make_segsum(D_HID)


BLK = 2048
NBLK = NPAD // BLK

_row_spec = pl.BlockSpec((BLK, D_HID), lambda i: (i, 0))
_col_spec = pl.BlockSpec((BLK, 1), lambda i: (i, 0))
_vec128_spec = pl.BlockSpec((D_HID,), lambda i: (0,))


def _tc_mm_body(x_ref, w0_ref, xw_ref):
    xw_ref[...] = jnp.dot(x_ref[...], w0_ref[...],
                          preferred_element_type=jnp.float32,
                          precision=lax.Precision.HIGHEST)


def _tc_mm(x_pad, w0):
    return pl.pallas_call(
        _tc_mm_body,
        grid=(NBLK,),
        in_specs=[_row_spec, pl.BlockSpec((D_IN, D_HID), lambda i: (0, 0))],
        out_specs=_row_spec,
        out_shape=jax.ShapeDtypeStruct((NPAD, D_HID), jnp.float32),
    )(x_pad, w0)


def _tc_a_body(hist_ref, xw_ref, y0_ref, indeg_ref):
    ones = jnp.ones((NW, 1), jnp.float32)
    indeg = lax.dot_general(hist_ref[...], ones, (((0,), (0,)), ((), ())),
                            preferred_element_type=jnp.float32)
    dinv = lax.rsqrt(indeg + 1.0)
    y0_ref[...] = xw_ref[...] * dinv
    indeg_ref[...] = indeg


def _tc_a(hist, xw):
    return pl.pallas_call(
        _tc_a_body,
        grid=(NBLK,),
        in_specs=[
            pl.BlockSpec((NW, BLK), lambda i: (0, i)),
            _row_spec,
        ],
        out_specs=[_row_spec, _col_spec],
        out_shape=[
            jax.ShapeDtypeStruct((NPAD, D_HID), jnp.float32),
            jax.ShapeDtypeStruct((NPAD, 1), jnp.float32),
        ],
    )(hist, xw)


def _tc_b_body(p_ref, xw_ref, indeg_ref, b0_ref, g0_ref, be0_ref, w1_ref,
               y1_ref, hw1_ref, c1_ref, c2_ref, h_s, mm_s):
    ph = pl.program_id(0)
    i = pl.program_id(1)
    rows = i * BLK + lax.broadcasted_iota(jnp.int32, (BLK, 1), 0)
    valid = rows < N
    indeg = indeg_ref[...]

    @pl.when(ph == 0)
    def _():
        dinv = lax.rsqrt(indeg + 1.0)
        agg = ((p_ref[0] + p_ref[1]) * dinv
               + dinv * dinv * xw_ref[...] + b0_ref[...][None, :])
        mu = jnp.mean(agg, axis=1, keepdims=True)
        cen = agg - mu
        var = jnp.mean(cen * cen, axis=1, keepdims=True)
        h = (cen * lax.rsqrt(var + 1e-5) * g0_ref[...][None, :]
             + be0_ref[...][None, :])
        h = jnp.maximum(h, 0.0)
        h_s[pl.ds(i * BLK, BLK), :] = h
        scores = jnp.sqrt(jnp.sum(h * h, axis=1, keepdims=True))
        big = jnp.float32(3e38)
        mm_s[0, i] = jnp.min(jnp.where(valid, scores, big))
        mm_s[1, i] = jnp.max(jnp.where(valid, scores, -big))
        y1_ref[...] = jnp.zeros((BLK, D_HID), jnp.float32)
        hw1_ref[...] = jnp.zeros((BLK, D_OUT), jnp.float32)
        c1_ref[...] = jnp.zeros((BLK, 1), jnp.float32)
        c2_ref[...] = jnp.zeros((BLK, 1), jnp.float32)

    @pl.when(ph == 1)
    def _():
        smin = mm_s[0, 0]
        smax = mm_s[1, 0]
        for k in range(1, NBLK):
            smin = jnp.minimum(smin, mm_s[0, k])
            smax = jnp.maximum(smax, mm_s[1, k])
        h = h_s[pl.ds(i * BLK, BLK), :]
        scores = jnp.sqrt(jnp.sum(h * h, axis=1, keepdims=True))
        scn = (scores - smin) / (smax - smin + 1e-8)
        m = jnp.where(jnp.logical_and(scn > LCS, valid), 1.0, 0.0)
        dinv2 = lax.rsqrt(m * indeg + 1.0)
        hw1 = jnp.dot(h, w1_ref[...],
                      preferred_element_type=jnp.float32,
                      precision=lax.Precision.HIGHEST)
        y1 = hw1 * dinv2
        y1_ref[...] = jnp.concatenate(
            [y1, jnp.zeros((BLK, D_HID - D_OUT), jnp.float32)], axis=1)
        hw1_ref[...] = hw1
        c1_ref[...] = m * dinv2
        c2_ref[...] = dinv2 * dinv2


def _tc_b(p, xw, indeg, b0, g0, be0, w1):
    return pl.pallas_call(
        _tc_b_body,
        grid=(2, NBLK),
        in_specs=[pl.BlockSpec((2, BLK, D_HID), lambda ph, i: (0, i, 0)),
                  pl.BlockSpec((BLK, D_HID), lambda ph, i: (i, 0)),
                  pl.BlockSpec((BLK, 1), lambda ph, i: (i, 0)),
                  pl.BlockSpec((D_HID,), lambda ph, i: (0,)),
                  pl.BlockSpec((D_HID,), lambda ph, i: (0,)),
                  pl.BlockSpec((D_HID,), lambda ph, i: (0,)),
                  pl.BlockSpec((D_HID, D_OUT), lambda ph, i: (0, 0))],
        out_specs=[pl.BlockSpec((BLK, D_HID), lambda ph, i: (i, 0)),
                   pl.BlockSpec((BLK, D_OUT), lambda ph, i: (i, 0)),
                   pl.BlockSpec((BLK, 1), lambda ph, i: (i, 0)),
                   pl.BlockSpec((BLK, 1), lambda ph, i: (i, 0))],
        out_shape=[
            jax.ShapeDtypeStruct((NPAD, D_HID), jnp.float32),
            jax.ShapeDtypeStruct((NPAD, D_OUT), jnp.float32),
            jax.ShapeDtypeStruct((NPAD, 1), jnp.float32),
            jax.ShapeDtypeStruct((NPAD, 1), jnp.float32),
        ],
        scratch_shapes=[
            pltpu.VMEM((NPAD, D_HID), jnp.float32),
            pltpu.SMEM((2, NBLK), jnp.float32),
        ],
    )(p, xw, indeg, b0, g0, be0, w1)


def _tc_c_body(q_ref, hw1_ref, c1_ref, c2_ref, b1_ref, out_ref):
    q = q_ref[0] + q_ref[1]
    o = (q[:, :D_OUT] * c1_ref[...]
         + c2_ref[...] * hw1_ref[...] + b1_ref[...][None, :])
    z = o - jnp.max(o, axis=1, keepdims=True)
    out_ref[...] = z - jnp.log(jnp.sum(jnp.exp(z), axis=1, keepdims=True))


def _tc_c(q, hw1, c1, c2, b1):
    return pl.pallas_call(
        _tc_c_body,
        grid=(NBLK,),
        in_specs=[pl.BlockSpec((2, BLK, D_HID), lambda i: (0, i, 0)),
                  pl.BlockSpec((BLK, D_OUT), lambda i: (i, 0)),
                  _col_spec, _col_spec,
                  pl.BlockSpec((D_OUT,), lambda i: (0,))],
        out_specs=pl.BlockSpec((BLK, D_OUT), lambda i: (i, 0)),
        out_shape=jax.ShapeDtypeStruct((NPAD, D_OUT), jnp.float32),
    )(q, hw1, c1, c2, b1)


def kernel(x, edge_index, W0, b0, gamma0, beta0, W1, b1):
    nds = EPP - EPW
    ds_ = jnp.arange(nds, dtype=jnp.int32)
    dum_src = jnp.broadcast_to((ds_ * 97) % N, (NW, nds))
    ndd = EPP - EPW - CHUNK
    dd_ = jnp.arange(ndd, dtype=jnp.int32)
    dum_dst = jnp.broadcast_to(N + dd_ % (NPAD - N), (NW, ndd))
    ddl = jnp.arange(CHUNK, dtype=jnp.int32)
    dum_dst0 = jnp.broadcast_to(N + ddl % (NPAD - N), (NW, CHUNK))
    srcA = jnp.concatenate(
        [edge_index[0].reshape(NW, EPW), dum_src], axis=1
    ).reshape(NW, 1, NROW, UROW)
    dstA = jnp.concatenate(
        [dum_dst0, edge_index[1].reshape(NW, EPW), dum_dst], axis=1
    ).reshape(NW, 1, NROW, UROW)
    edges = jnp.concatenate([srcA, dstA], axis=1)
    dst_flat = edge_index[1].reshape(NW, EPW)
    x_pad = jnp.pad(x, ((0, NPAD - N), (0, 0)))

    hist = _sc_hist(dst_flat)
    xw = _tc_mm(x_pad, W0)
    y0, indeg = _tc_a(hist, xw)
    parts0 = _sc_segsum_hid(y0, edges)
    y1, hw1, c1, c2 = _tc_b(parts0, xw, indeg, b0, gamma0, beta0, W1)
    parts1 = _sc_segsum_hid(y1, edges)
    out = _tc_c(parts1, hw1, c1, c2, b1)
    return out[:N]
